# baseline (speedup 1.0000x reference)
import functools

import jax
import jax.numpy as jnp
from jax import lax
from jax.experimental import pallas as pl
from jax.experimental.pallas import tpu as pltpu
from jax.experimental.pallas import tpu_sc as plsc

_B = 2
_N = 2048
_C = 128
_L = 16
_G = _C // _L
_R = 256
_NBLK = _N // _R

_mesh = plsc.VectorSubcoreMesh(core_axis_name="c", subcore_axis_name="s")


@functools.partial(
    pl.kernel,
    out_type=jax.ShapeDtypeStruct((_B, _N, _N), jnp.float32),
    mesh=_mesh,
    scratch_types=[pltpu.VMEM((_R, _C), jnp.float32)],
)
def _sc_suffix_sum(x_hbm, o_hbm, buf):
    wid = lax.axis_index("s") * 2 + lax.axis_index("c")
    b = wid // (_N // _C)
    c0 = (wid % (_N // _C)) * _C

    carries = tuple(jnp.zeros((_L,), jnp.float32) for _ in range(_G))
    for blk in range(_NBLK):
        r0 = _N - (blk + 1) * _R
        pltpu.sync_copy(x_hbm.at[b, pl.ds(r0, _R), pl.ds(c0, _C)], buf)

        def row_body(t, carry):
            i = _R - 1 - t
            new = []
            for g in range(_G):
                v = buf[i, pl.ds(g * _L, _L)] + carry[g]
                buf[i, pl.ds(g * _L, _L)] = v
                new.append(v)
            return tuple(new)

        carries = lax.fori_loop(0, _R, row_body, carries)
        pltpu.sync_copy(buf, o_hbm.at[b, pl.ds(r0, _R), pl.ds(c0, _C)])


def kernel(x):
    return _sc_suffix_sum(x)


# speedup vs baseline: 1.2888x; 1.2888x over previous
import functools

import jax
import jax.numpy as jnp
from jax import lax
from jax.experimental import pallas as pl
from jax.experimental.pallas import tpu as pltpu
from jax.experimental.pallas import tpu_sc as plsc

_B = 2
_N = 2048
_C = 128
_L = 16
_G = _C // _L
_R = 256
_NBLK = _N // _R
_NBUF = 3

_mesh = plsc.VectorSubcoreMesh(core_axis_name="c", subcore_axis_name="s")


@functools.partial(
    pl.kernel,
    out_type=jax.ShapeDtypeStruct((_B, _N, _N), jnp.float32),
    mesh=_mesh,
    scratch_types=[
        pltpu.VMEM((_NBUF, _R, _C), jnp.float32),
        pltpu.SemaphoreType.DMA((_NBUF,)),
        pltpu.SemaphoreType.DMA((_NBUF,)),
    ],
)
def _sc_suffix_sum(x_hbm, o_hbm, buf, in_sems, out_sems):
    wid = lax.axis_index("s") * 2 + lax.axis_index("c")
    b = wid // (_N // _C)
    c0 = (wid % (_N // _C)) * _C

    def start_in(blk, s):
        r0 = _N - (blk + 1) * _R
        return pltpu.async_copy(
            x_hbm.at[b, pl.ds(r0, _R), pl.ds(c0, _C)], buf.at[s], in_sems.at[s]
        )

    def start_out(blk, s):
        r0 = _N - (blk + 1) * _R
        return pltpu.async_copy(
            buf.at[s], o_hbm.at[b, pl.ds(r0, _R), pl.ds(c0, _C)], out_sems.at[s]
        )

    copies_in = {0: start_in(0, 0)}
    copies_out = {}
    carries = tuple(jnp.zeros((_L,), jnp.float32) for _ in range(_G))
    for blk in range(_NBLK):
        s = blk % _NBUF
        copies_in[blk].wait()
        if blk + 1 < _NBLK:
            sn = (blk + 1) % _NBUF
            if blk + 1 - _NBUF >= 0:
                copies_out[blk + 1 - _NBUF].wait()
            copies_in[blk + 1] = start_in(blk + 1, sn)

        def row_body(t, carry):
            i = _R - 1 - t
            new = []
            for g in range(_G):
                v = buf[s, i, pl.ds(g * _L, _L)] + carry[g]
                buf[s, i, pl.ds(g * _L, _L)] = v
                new.append(v)
            return tuple(new)

        carries = lax.fori_loop(0, _R, row_body, carries)
        copies_out[blk] = start_out(blk, s)

    for blk in range(max(0, _NBLK - _NBUF), _NBLK):
        copies_out[blk].wait()


def kernel(x):
    return _sc_suffix_sum(x)
